# baseline (speedup 1.0000x reference)
import jax
import jax.numpy as jnp
from jax import lax
from jax.experimental import pallas as pl
from jax.experimental.pallas import tpu as pltpu
from jax.experimental.pallas import tpu_sc as plsc

_B = 16384
_D = 64
_DP = 128
_NPAIR = 50000
_NC = 2
_NS = 16
_NW = _NC * _NS
_BPW = _B // _NW
_CH = 128
_NCHUNK = _BPW // _CH


def _sc_gather_body(u_idx_hbm, i_idx_hbm, gu_hbm, gi_hbm, mu_hbm, mi_hbm,
                    gu_out, gi_out, mu_out, mi_out,
                    pidx_u, pidx_i, bb0, bb1, gsem0, gsem1, ssem0, ssem1):
  wid = lax.axis_index("s") * _NC + lax.axis_index("c")
  base = wid * _BPW
  pltpu.sync_copy(u_idx_hbm.at[pl.ds(base, _BPW)], pidx_u)
  pltpu.sync_copy(i_idx_hbm.at[pl.ds(base, _BPW)], pidx_i)
  for pidx in (pidx_u, pidx_i):
    for g in range(_BPW // 16):
      pidx[pl.ds(g * 16, 16)] = lax.shift_right_logical(
          pidx[pl.ds(g * 16, 16)], 1)

  plan = []
  for tab, pidx, out in ((gu_hbm, pidx_u, gu_out), (gi_hbm, pidx_i, gi_out),
                         (mu_hbm, pidx_u, mu_out), (mi_hbm, pidx_i, mi_out)):
    for c in range(_NCHUNK):
      plan.append((tab, pidx, out, c))

  bbufs = (bb0, bb1)
  gsems = (gsem0, gsem1)
  ssems = (ssem0, ssem1)

  def start_gather(n):
    tab, pidx, _, c = plan[n]
    return pltpu.async_copy(
        tab.at[pidx.at[pl.ds(c * _CH, _CH)]], bbufs[n % 2], gsems[n % 2])

  def write_back(n):
    _, _, out, c = plan[n]
    return pltpu.async_copy(
        bbufs[n % 2], out.at[pl.ds(base + c * _CH, _CH)], ssems[n % 2])

  wb = [None, None]
  g = [start_gather(0), None]
  for n in range(len(plan)):
    par = n % 2
    if n + 1 < len(plan):
      if wb[1 - par] is not None:
        wb[1 - par].wait()
      g[1 - par] = start_gather(n + 1)
    g[par].wait()
    wb[par] = write_back(n)
  wb[0].wait()
  wb[1].wait()


def _sc_gather(u_idx, i_idx, gu_t, gi_t, mu_t, mi_t):
  mesh = plsc.VectorSubcoreMesh(core_axis_name="c", subcore_axis_name="s",
                                num_cores=_NC, num_subcores=_NS)
  emb = jax.ShapeDtypeStruct((_B, _DP), jnp.float32)
  f = pl.kernel(
      _sc_gather_body,
      out_type=[emb, emb, emb, emb],
      mesh=mesh,
      scratch_types=[
          pltpu.VMEM((_BPW,), jnp.int32),
          pltpu.VMEM((_BPW,), jnp.int32),
          pltpu.VMEM((_CH, _DP), jnp.float32),
          pltpu.VMEM((_CH, _DP), jnp.float32),
          pltpu.SemaphoreType.DMA,
          pltpu.SemaphoreType.DMA,
          pltpu.SemaphoreType.DMA,
          pltpu.SemaphoreType.DMA,
      ],
  )
  return f(u_idx, i_idx, gu_t, gi_t, mu_t, mi_t)


_BB = 2048


def _tc_mlp_body(uidx, iidx, gup, gip, mup, mip, w1u, w1i, b1, w2, b2, w3, b3,
                 wpg, wph, bp, out):
  usel = (uidx[...] & 1) == 1
  isel = (iidx[...] & 1) == 1
  gu = jnp.where(usel, gup[:, _D:], gup[:, :_D])
  mu = jnp.where(usel, mup[:, _D:], mup[:, :_D])
  gi = jnp.where(isel, gip[:, _D:], gip[:, :_D])
  mi = jnp.where(isel, mip[:, _D:], mip[:, :_D])
  h = jnp.dot(mu, w1u[...], preferred_element_type=jnp.float32)
  h += jnp.dot(mi, w1i[...], preferred_element_type=jnp.float32)
  h = jnp.maximum(h + b1[...], 0.0)
  h = jnp.maximum(
      jnp.dot(h, w2[...], preferred_element_type=jnp.float32) + b2[...], 0.0)
  h = jnp.maximum(
      jnp.dot(h, w3[...], preferred_element_type=jnp.float32) + b3[...], 0.0)
  pred = jnp.dot(gu * gi, wpg[...], preferred_element_type=jnp.float32)
  pred += jnp.dot(h, wph[...], preferred_element_type=jnp.float32)
  out[...] = pred + bp[...]


def _tc_mlp(uidx, iidx, gup, gip, mup, mip, w1u, w1i, b1, w2, b2, w3, b3,
            wpg, wph, bp):
  grid = (_B // _BB,)
  emb_spec = pl.BlockSpec((_BB, _DP), lambda i: (i, 0))
  idx_spec = pl.BlockSpec((_BB, 1), lambda i: (i, 0))

  def full(shape):
    return pl.BlockSpec(shape, lambda i: (0,) * len(shape))

  return pl.pallas_call(
      _tc_mlp_body,
      grid=grid,
      in_specs=[
          idx_spec, idx_spec,
          emb_spec, emb_spec, emb_spec, emb_spec,
          full(w1u.shape), full(w1i.shape), full(b1.shape),
          full(w2.shape), full(b2.shape),
          full(w3.shape), full(b3.shape),
          full(wpg.shape), full(wph.shape), full(bp.shape),
      ],
      out_specs=pl.BlockSpec((_BB, 1), lambda i: (i, 0)),
      out_shape=jax.ShapeDtypeStruct((_B, 1), jnp.float32),
  )(uidx, iidx, gup, gip, mup, mip, w1u, w1i, b1, w2, b2, w3, b3, wpg, wph, bp)


def kernel(user_indices, item_indices, gmf_user, gmf_item, mlp_user, mlp_item,
           W1, b1, W2, b2, W3, b3, Wp, bp):
  user_indices = user_indices.astype(jnp.int32)
  item_indices = item_indices.astype(jnp.int32)
  gup, gip, mup, mip = _sc_gather(
      user_indices, item_indices,
      gmf_user.reshape(_NPAIR, _DP), gmf_item.reshape(_NPAIR, _DP),
      mlp_user.reshape(_NPAIR, _DP), mlp_item.reshape(_NPAIR, _DP))
  w1u = W1[:_D, :]
  w1i = W1[_D:, :]
  wpg = Wp[:_D, :]
  wph = Wp[_D:, :]
  pred = _tc_mlp(user_indices.reshape(_B, 1), item_indices.reshape(_B, 1),
                 gup, gip, mup, mip,
                 w1u, w1i, b1.reshape(1, -1), W2, b2.reshape(1, -1),
                 W3, b3.reshape(1, -1), wpg, wph, bp.reshape(1, 1))
  return pred[:, 0]


# speedup vs baseline: 1.7578x; 1.7578x over previous
import jax
import jax.numpy as jnp
from jax import lax
from jax.experimental import pallas as pl
from jax.experimental.pallas import tpu as pltpu
from jax.experimental.pallas import tpu_sc as plsc

_B = 16384
_D = 64
_NC = 2
_NS = 16
_NW = _NC * _NS
_BPW = _B // _NW
_CH = 64
_NCHUNK = _BPW // _CH


def _sc_gather_body(u_idx_hbm, i_idx_hbm, gmf_u_hbm, gmf_i_hbm, mlp_u_hbm,
                    mlp_i_hbm, gu_out, gi_out, mu_out, mi_out,
                    idx_u_v, idx_i_v, idx_u_s, idx_i_s,
                    ob0, ob1, gsem0, gsem1, ssem0, ssem1):
  wid = lax.axis_index("s") * _NC + lax.axis_index("c")
  base = wid * _BPW
  pltpu.sync_copy(u_idx_hbm.at[pl.ds(base, _BPW)], idx_u_v)
  pltpu.sync_copy(i_idx_hbm.at[pl.ds(base, _BPW)], idx_i_v)

  for idx_v, idx_s in ((idx_u_v, idx_u_s), (idx_i_v, idx_i_s)):
    for g in range(_BPW // 16):
      v = idx_v[pl.ds(g * 16, 16)]
      for l in range(16):
        idx_s[g * 16 + l] = v[l]

  plan = []
  for tab, idx_s, out in (
      (gmf_u_hbm, idx_u_s, gu_out),
      (gmf_i_hbm, idx_i_s, gi_out),
      (mlp_u_hbm, idx_u_s, mu_out),
      (mlp_i_hbm, idx_i_s, mi_out)):
    for c in range(_NCHUNK):
      plan.append((tab, idx_s, out, c))

  obufs = (ob0, ob1)
  gsems = (gsem0, gsem1)
  ssems = (ssem0, ssem1)

  def issue_rows(n):
    tab, idx_s, _, c = plan[n]
    par = n % 2

    def body(j, _):
      v = idx_s[c * _CH + j]
      b = lax.shift_right_logical(v, 3)
      r = v & 7
      pltpu.async_copy(tab.at[b, r], obufs[par].at[j], gsems[par])
      return 0

    lax.fori_loop(0, _CH, body, 0, unroll=4)

  def drain_rows(n):
    par = n % 2
    out = plan[n][2]
    pltpu.make_async_copy(out.at[pl.ds(0, _CH)], obufs[par], gsems[par]).wait()

  def write_back(n):
    _, _, out, c = plan[n]
    return pltpu.async_copy(
        obufs[n % 2], out.at[pl.ds(base + c * _CH, _CH)], ssems[n % 2])

  wb = [None, None]
  for n in range(len(plan)):
    par = n % 2
    if wb[par] is not None:
      wb[par].wait()
    issue_rows(n)
    if n >= 1:
      drain_rows(n - 1)
      wb[(n - 1) % 2] = write_back(n - 1)
  last = len(plan) - 1
  drain_rows(last)
  wb[last % 2] = write_back(last)
  wb[0].wait()
  wb[1].wait()


def _sc_gather(u_idx, i_idx, gmf_u, gmf_i, mlp_u, mlp_i):
  mesh = plsc.VectorSubcoreMesh(core_axis_name="c", subcore_axis_name="s",
                                num_cores=_NC, num_subcores=_NS)
  emb = jax.ShapeDtypeStruct((_B, _D), jnp.float32)
  f = pl.kernel(
      _sc_gather_body,
      out_type=[emb, emb, emb, emb],
      mesh=mesh,
      scratch_types=[
          pltpu.VMEM((_BPW,), jnp.int32),
          pltpu.VMEM((_BPW,), jnp.int32),
          pltpu.SMEM((_BPW,), jnp.int32),
          pltpu.SMEM((_BPW,), jnp.int32),
          pltpu.VMEM((_CH, _D), jnp.float32),
          pltpu.VMEM((_CH, _D), jnp.float32),
          pltpu.SemaphoreType.DMA,
          pltpu.SemaphoreType.DMA,
          pltpu.SemaphoreType.DMA,
          pltpu.SemaphoreType.DMA,
      ],
  )
  return f(u_idx, i_idx, gmf_u, gmf_i, mlp_u, mlp_i)


_BB = 2048


def _tc_mlp_body(gu_r, gi_r, mu_r, mi_r, w1u, w1i, b1, w2, b2, w3, b3, wpg,
                 wph, bp, out):
  gu = gu_r[...]
  mu = mu_r[...]
  gi = gi_r[...]
  mi = mi_r[...]
  h = jnp.dot(mu, w1u[...], preferred_element_type=jnp.float32)
  h += jnp.dot(mi, w1i[...], preferred_element_type=jnp.float32)
  h = jnp.maximum(h + b1[...], 0.0)
  h = jnp.maximum(
      jnp.dot(h, w2[...], preferred_element_type=jnp.float32) + b2[...], 0.0)
  h = jnp.maximum(
      jnp.dot(h, w3[...], preferred_element_type=jnp.float32) + b3[...], 0.0)
  pred = jnp.dot(gu * gi, wpg[...], preferred_element_type=jnp.float32)
  pred += jnp.dot(h, wph[...], preferred_element_type=jnp.float32)
  out[...] = pred + bp[...]


def _tc_mlp(gu, gi, mu, mi, w1u, w1i, b1, w2, b2, w3, b3, wpg, wph, bp):
  grid = (_B // _BB,)
  emb_spec = pl.BlockSpec((_BB, _D), lambda i: (i, 0))

  def full(shape):
    return pl.BlockSpec(shape, lambda i: (0,) * len(shape))

  return pl.pallas_call(
      _tc_mlp_body,
      grid=grid,
      in_specs=[
          emb_spec, emb_spec, emb_spec, emb_spec,
          full(w1u.shape), full(w1i.shape), full(b1.shape),
          full(w2.shape), full(b2.shape),
          full(w3.shape), full(b3.shape),
          full(wpg.shape), full(wph.shape), full(bp.shape),
      ],
      out_specs=pl.BlockSpec((_BB, 1), lambda i: (i, 0)),
      out_shape=jax.ShapeDtypeStruct((_B, 1), jnp.float32),
  )(gu, gi, mu, mi, w1u, w1i, b1, w2, b2, w3, b3, wpg, wph, bp)


def kernel(user_indices, item_indices, gmf_user, gmf_item, mlp_user, mlp_item,
           W1, b1, W2, b2, W3, b3, Wp, bp):
  user_indices = user_indices.astype(jnp.int32)
  item_indices = item_indices.astype(jnp.int32)
  gu, gi, mu, mi = _sc_gather(
      user_indices, item_indices,
      gmf_user.reshape(12500, 8, _D), gmf_item.reshape(12500, 8, _D),
      mlp_user.reshape(12500, 8, _D), mlp_item.reshape(12500, 8, _D))
  w1u = W1[:_D, :]
  w1i = W1[_D:, :]
  wpg = Wp[:_D, :]
  wph = Wp[_D:, :]
  pred = _tc_mlp(gu, gi, mu, mi, w1u, w1i, b1.reshape(1, -1),
                 W2, b2.reshape(1, -1), W3, b3.reshape(1, -1),
                 wpg, wph, bp.reshape(1, 1))
  return pred[:, 0]


# speedup vs baseline: 1.8209x; 1.0359x over previous
import jax
import jax.numpy as jnp
from jax import lax
from jax.experimental import pallas as pl
from jax.experimental.pallas import tpu as pltpu
from jax.experimental.pallas import tpu_sc as plsc

_B = 16384
_D = 64
_NC = 2
_NS = 16
_NW = _NC * _NS
_BPW = _B // _NW
_CH = 128
_NCHUNK = _BPW // _CH


def _sc_gather_body(u_idx_hbm, i_idx_hbm, gmf_u_hbm, gmf_i_hbm, mlp_u_hbm,
                    mlp_i_hbm, gu_out, gi_out, mu_out, mi_out,
                    idx_u_v, idx_i_v, idx_u_s, idx_i_s,
                    ob0, ob1, gsem0, gsem1, ssem0, ssem1):
  wid = lax.axis_index("s") * _NC + lax.axis_index("c")
  base = wid * _BPW
  pltpu.sync_copy(u_idx_hbm.at[pl.ds(base, _BPW)], idx_u_v)
  pltpu.sync_copy(i_idx_hbm.at[pl.ds(base, _BPW)], idx_i_v)

  for idx_v, idx_s in ((idx_u_v, idx_u_s), (idx_i_v, idx_i_s)):
    for g in range(_BPW // 16):
      v = idx_v[pl.ds(g * 16, 16)]
      for l in range(16):
        idx_s[g * 16 + l] = v[l]

  plan = []
  for tab, idx_s, out in (
      (gmf_u_hbm, idx_u_s, gu_out),
      (gmf_i_hbm, idx_i_s, gi_out),
      (mlp_u_hbm, idx_u_s, mu_out),
      (mlp_i_hbm, idx_i_s, mi_out)):
    for c in range(_NCHUNK):
      plan.append((tab, idx_s, out, c))

  obufs = (ob0, ob1)
  gsems = (gsem0, gsem1)
  ssems = (ssem0, ssem1)

  def issue_rows(n):
    tab, idx_s, _, c = plan[n]
    par = n % 2

    def body(j, _):
      v = idx_s[c * _CH + j]
      b = lax.shift_right_logical(v, 3)
      r = v & 7
      pltpu.async_copy(tab.at[b, r], obufs[par].at[j], gsems[par])
      return 0

    lax.fori_loop(0, _CH, body, 0, unroll=4)

  def drain_rows(n):
    par = n % 2
    out = plan[n][2]
    pltpu.make_async_copy(out.at[pl.ds(0, _CH)], obufs[par], gsems[par]).wait()

  def write_back(n):
    _, _, out, c = plan[n]
    return pltpu.async_copy(
        obufs[n % 2], out.at[pl.ds(base + c * _CH, _CH)], ssems[n % 2])

  wb = [None, None]
  for n in range(len(plan)):
    par = n % 2
    if wb[par] is not None:
      wb[par].wait()
    issue_rows(n)
    if n >= 1:
      drain_rows(n - 1)
      wb[(n - 1) % 2] = write_back(n - 1)
  last = len(plan) - 1
  drain_rows(last)
  wb[last % 2] = write_back(last)
  wb[0].wait()
  wb[1].wait()


def _sc_gather(u_idx, i_idx, gmf_u, gmf_i, mlp_u, mlp_i):
  mesh = plsc.VectorSubcoreMesh(core_axis_name="c", subcore_axis_name="s",
                                num_cores=_NC, num_subcores=_NS)
  emb = jax.ShapeDtypeStruct((_B, _D), jnp.float32)
  f = pl.kernel(
      _sc_gather_body,
      out_type=[emb, emb, emb, emb],
      mesh=mesh,
      scratch_types=[
          pltpu.VMEM((_BPW,), jnp.int32),
          pltpu.VMEM((_BPW,), jnp.int32),
          pltpu.SMEM((_BPW,), jnp.int32),
          pltpu.SMEM((_BPW,), jnp.int32),
          pltpu.VMEM((_CH, _D), jnp.float32),
          pltpu.VMEM((_CH, _D), jnp.float32),
          pltpu.SemaphoreType.DMA,
          pltpu.SemaphoreType.DMA,
          pltpu.SemaphoreType.DMA,
          pltpu.SemaphoreType.DMA,
      ],
  )
  return f(u_idx, i_idx, gmf_u, gmf_i, mlp_u, mlp_i)


_BB = 2048


def _tc_mlp_body(gu_r, gi_r, mu_r, mi_r, w1u, w1i, b1, w2, b2, w3, b3, wpg,
                 wph, bp, out):
  gu = gu_r[...]
  mu = mu_r[...]
  gi = gi_r[...]
  mi = mi_r[...]
  h = jnp.dot(mu, w1u[...], preferred_element_type=jnp.float32)
  h += jnp.dot(mi, w1i[...], preferred_element_type=jnp.float32)
  h = jnp.maximum(h + b1[...], 0.0)
  h = jnp.maximum(
      jnp.dot(h, w2[...], preferred_element_type=jnp.float32) + b2[...], 0.0)
  h = jnp.maximum(
      jnp.dot(h, w3[...], preferred_element_type=jnp.float32) + b3[...], 0.0)
  pred = jnp.dot(gu * gi, wpg[...], preferred_element_type=jnp.float32)
  pred += jnp.dot(h, wph[...], preferred_element_type=jnp.float32)
  out[...] = pred + bp[...]


def _tc_mlp(gu, gi, mu, mi, w1u, w1i, b1, w2, b2, w3, b3, wpg, wph, bp):
  grid = (_B // _BB,)
  emb_spec = pl.BlockSpec((_BB, _D), lambda i: (i, 0))

  def full(shape):
    return pl.BlockSpec(shape, lambda i: (0,) * len(shape))

  return pl.pallas_call(
      _tc_mlp_body,
      grid=grid,
      in_specs=[
          emb_spec, emb_spec, emb_spec, emb_spec,
          full(w1u.shape), full(w1i.shape), full(b1.shape),
          full(w2.shape), full(b2.shape),
          full(w3.shape), full(b3.shape),
          full(wpg.shape), full(wph.shape), full(bp.shape),
      ],
      out_specs=pl.BlockSpec((_BB, 1), lambda i: (i, 0)),
      out_shape=jax.ShapeDtypeStruct((_B, 1), jnp.float32),
  )(gu, gi, mu, mi, w1u, w1i, b1, w2, b2, w3, b3, wpg, wph, bp)


def kernel(user_indices, item_indices, gmf_user, gmf_item, mlp_user, mlp_item,
           W1, b1, W2, b2, W3, b3, Wp, bp):
  user_indices = user_indices.astype(jnp.int32)
  item_indices = item_indices.astype(jnp.int32)
  gu, gi, mu, mi = _sc_gather(
      user_indices, item_indices,
      gmf_user.reshape(12500, 8, _D), gmf_item.reshape(12500, 8, _D),
      mlp_user.reshape(12500, 8, _D), mlp_item.reshape(12500, 8, _D))
  w1u = W1[:_D, :]
  w1i = W1[_D:, :]
  wpg = Wp[:_D, :]
  wph = Wp[_D:, :]
  pred = _tc_mlp(gu, gi, mu, mi, w1u, w1i, b1.reshape(1, -1),
                 W2, b2.reshape(1, -1), W3, b3.reshape(1, -1),
                 wpg, wph, bp.reshape(1, 1))
  return pred[:, 0]


# speedup vs baseline: 1.8353x; 1.0079x over previous
import jax
import jax.numpy as jnp
from jax import lax
from jax.experimental import pallas as pl
from jax.experimental.pallas import tpu as pltpu
from jax.experimental.pallas import tpu_sc as plsc

_B = 16384
_D = 64
_NC = 2
_NS = 16
_NW = _NC * _NS
_BPW = _B // _NW
_CH = 256
_NCHUNK = _BPW // _CH


def _sc_gather_body(u_idx_hbm, i_idx_hbm, gmf_u_hbm, gmf_i_hbm, mlp_u_hbm,
                    mlp_i_hbm, gu_out, gi_out, mu_out, mi_out,
                    idx_u_v, idx_i_v, idx_u_s, idx_i_s,
                    ob0, ob1, gsem0, gsem1, ssem0, ssem1):
  wid = lax.axis_index("s") * _NC + lax.axis_index("c")
  base = wid * _BPW
  pltpu.sync_copy(u_idx_hbm.at[pl.ds(base, _BPW)], idx_u_v)
  pltpu.sync_copy(i_idx_hbm.at[pl.ds(base, _BPW)], idx_i_v)

  for idx_v, idx_s in ((idx_u_v, idx_u_s), (idx_i_v, idx_i_s)):
    for g in range(_BPW // 16):
      v = idx_v[pl.ds(g * 16, 16)]
      for l in range(16):
        idx_s[g * 16 + l] = v[l]

  plan = []
  for tab, idx_s, out in (
      (gmf_u_hbm, idx_u_s, gu_out),
      (gmf_i_hbm, idx_i_s, gi_out),
      (mlp_u_hbm, idx_u_s, mu_out),
      (mlp_i_hbm, idx_i_s, mi_out)):
    for c in range(_NCHUNK):
      plan.append((tab, idx_s, out, c))

  obufs = (ob0, ob1)
  gsems = (gsem0, gsem1)
  ssems = (ssem0, ssem1)

  def issue_rows(n):
    tab, idx_s, _, c = plan[n]
    par = n % 2

    def body(j, _):
      v = idx_s[c * _CH + j]
      b = lax.shift_right_logical(v, 3)
      r = v & 7
      pltpu.async_copy(tab.at[b, r], obufs[par].at[j], gsems[par])
      return 0

    lax.fori_loop(0, _CH, body, 0, unroll=4)

  def drain_rows(n):
    par = n % 2
    out = plan[n][2]
    pltpu.make_async_copy(out.at[pl.ds(0, _CH)], obufs[par], gsems[par]).wait()

  def write_back(n):
    _, _, out, c = plan[n]
    return pltpu.async_copy(
        obufs[n % 2], out.at[pl.ds(base + c * _CH, _CH)], ssems[n % 2])

  wb = [None, None]
  for n in range(len(plan)):
    par = n % 2
    if wb[par] is not None:
      wb[par].wait()
    issue_rows(n)
    if n >= 1:
      drain_rows(n - 1)
      wb[(n - 1) % 2] = write_back(n - 1)
  last = len(plan) - 1
  drain_rows(last)
  wb[last % 2] = write_back(last)
  wb[0].wait()
  wb[1].wait()


def _sc_gather(u_idx, i_idx, gmf_u, gmf_i, mlp_u, mlp_i):
  mesh = plsc.VectorSubcoreMesh(core_axis_name="c", subcore_axis_name="s",
                                num_cores=_NC, num_subcores=_NS)
  emb = jax.ShapeDtypeStruct((_B, _D), jnp.float32)
  f = pl.kernel(
      _sc_gather_body,
      out_type=[emb, emb, emb, emb],
      mesh=mesh,
      scratch_types=[
          pltpu.VMEM((_BPW,), jnp.int32),
          pltpu.VMEM((_BPW,), jnp.int32),
          pltpu.SMEM((_BPW,), jnp.int32),
          pltpu.SMEM((_BPW,), jnp.int32),
          pltpu.VMEM((_CH, _D), jnp.float32),
          pltpu.VMEM((_CH, _D), jnp.float32),
          pltpu.SemaphoreType.DMA,
          pltpu.SemaphoreType.DMA,
          pltpu.SemaphoreType.DMA,
          pltpu.SemaphoreType.DMA,
      ],
  )
  return f(u_idx, i_idx, gmf_u, gmf_i, mlp_u, mlp_i)


_BB = 2048


def _tc_mlp_body(gu_r, gi_r, mu_r, mi_r, w1u, w1i, b1, w2, b2, w3, b3, wpg,
                 wph, bp, out):
  gu = gu_r[...]
  mu = mu_r[...]
  gi = gi_r[...]
  mi = mi_r[...]
  h = jnp.dot(mu, w1u[...], preferred_element_type=jnp.float32)
  h += jnp.dot(mi, w1i[...], preferred_element_type=jnp.float32)
  h = jnp.maximum(h + b1[...], 0.0)
  h = jnp.maximum(
      jnp.dot(h, w2[...], preferred_element_type=jnp.float32) + b2[...], 0.0)
  h = jnp.maximum(
      jnp.dot(h, w3[...], preferred_element_type=jnp.float32) + b3[...], 0.0)
  pred = jnp.dot(gu * gi, wpg[...], preferred_element_type=jnp.float32)
  pred += jnp.dot(h, wph[...], preferred_element_type=jnp.float32)
  out[...] = pred + bp[...]


def _tc_mlp(gu, gi, mu, mi, w1u, w1i, b1, w2, b2, w3, b3, wpg, wph, bp):
  grid = (_B // _BB,)
  emb_spec = pl.BlockSpec((_BB, _D), lambda i: (i, 0))

  def full(shape):
    return pl.BlockSpec(shape, lambda i: (0,) * len(shape))

  return pl.pallas_call(
      _tc_mlp_body,
      grid=grid,
      in_specs=[
          emb_spec, emb_spec, emb_spec, emb_spec,
          full(w1u.shape), full(w1i.shape), full(b1.shape),
          full(w2.shape), full(b2.shape),
          full(w3.shape), full(b3.shape),
          full(wpg.shape), full(wph.shape), full(bp.shape),
      ],
      out_specs=pl.BlockSpec((_BB, 1), lambda i: (i, 0)),
      out_shape=jax.ShapeDtypeStruct((_B, 1), jnp.float32),
  )(gu, gi, mu, mi, w1u, w1i, b1, w2, b2, w3, b3, wpg, wph, bp)


def kernel(user_indices, item_indices, gmf_user, gmf_item, mlp_user, mlp_item,
           W1, b1, W2, b2, W3, b3, Wp, bp):
  user_indices = user_indices.astype(jnp.int32)
  item_indices = item_indices.astype(jnp.int32)
  gu, gi, mu, mi = _sc_gather(
      user_indices, item_indices,
      gmf_user.reshape(12500, 8, _D), gmf_item.reshape(12500, 8, _D),
      mlp_user.reshape(12500, 8, _D), mlp_item.reshape(12500, 8, _D))
  w1u = W1[:_D, :]
  w1i = W1[_D:, :]
  wpg = Wp[:_D, :]
  wph = Wp[_D:, :]
  pred = _tc_mlp(gu, gi, mu, mi, w1u, w1i, b1.reshape(1, -1),
                 W2, b2.reshape(1, -1), W3, b3.reshape(1, -1),
                 wpg, wph, bp.reshape(1, 1))
  return pred[:, 0]
